# Initial kernel scaffold; baseline (speedup 1.0000x reference)
import jax
import jax.numpy as jnp
from jax.experimental import pallas as pl
from jax.experimental.pallas import tpu as pltpu

BN = 512


def _gather_u_body(uidx_row_ref, e_ref, acc_ref):
    i = pl.program_id(0)
    bn, d = e_ref.shape
    b = uidx_row_ref.shape[1]
    m_ids = jax.lax.broadcasted_iota(jnp.int32, (bn, b), 0) + i * bn
    eq_t = (m_ids == uidx_row_ref[0, :][None, :]).astype(jnp.bfloat16)
    contrib = jax.lax.dot_general(
        eq_t, e_ref[...].astype(jnp.bfloat16),
        dimension_numbers=(((0,), (0,)), ((), ())),
        preferred_element_type=jnp.float32)

    @pl.when(i == 0)
    def _():
        acc_ref[...] = contrib

    @pl.when(i > 0)
    def _():
        acc_ref[...] += contrib


def _main_body(uidx_ref, iidx_ref, u_ref, r_ref, e_ref, out_ref,
               pnum_ref, pden_ref, avgu_ref):
    i = pl.program_id(0)
    nb = pl.num_programs(0)
    bn, ni = r_ref.shape
    b = u_ref.shape[0]

    r = r_ref[...]
    validb = jnp.logical_not(jnp.isnan(r))
    validf = validb.astype(jnp.float32)
    r0 = jnp.where(validb, r, 0.0)
    cnt = jnp.sum(validf, axis=1)
    ssum = jnp.sum(r0, axis=1)
    avg = jnp.where(cnt > 0.0, ssum / jnp.maximum(cnt, 1.0), 0.0)

    e = e_ref[...]
    nn = jnp.sqrt(jnp.sum(e * e, axis=1))
    u = u_ref[...]
    nu = jnp.sqrt(jnp.sum(u * u, axis=1))

    dots = jax.lax.dot_general(
        u.astype(jnp.bfloat16), e.astype(jnp.bfloat16),
        dimension_numbers=(((1,), (1,)), ((), ())),
        preferred_element_type=jnp.float32)
    sim = dots / (nu[:, None] * nn[None, :] + 1e-8)
    n_ids = jax.lax.broadcasted_iota(jnp.int32, (b, bn), 1) + i * bn
    eq = uidx_ref[...] == n_ids
    sim_x = jnp.where(eq, 0.0, sim)

    @pl.when(i == 0)
    def _():
        pnum_ref[...] = jnp.zeros_like(pnum_ref)
        pden_ref[...] = jnp.zeros_like(pden_ref)
        avgu_ref[...] = jnp.zeros_like(avgu_ref)

    avgu_ref[...] += jnp.sum(jnp.where(eq, avg[None, :], 0.0), axis=1)[:, None]

    acomb = (r0 - avg[:, None] * validf).astype(jnp.bfloat16)
    simx_bf = sim_x.astype(jnp.bfloat16)
    pnum_ref[...] += jax.lax.dot_general(
        simx_bf, acomb,
        dimension_numbers=(((1,), (0,)), ((), ())),
        preferred_element_type=jnp.float32)
    pden_ref[...] += jax.lax.dot_general(
        jnp.abs(simx_bf), validf.astype(jnp.bfloat16),
        dimension_numbers=(((1,), (0,)), ((), ())),
        preferred_element_type=jnp.float32)

    @pl.when(i == nb - 1)
    def _():
        j_ids = jax.lax.broadcasted_iota(jnp.int32, (b, ni), 1)
        sel = iidx_ref[...] == j_ids
        tnum = jnp.sum(jnp.where(sel, pnum_ref[...], 0.0), axis=1)
        tden = jnp.sum(jnp.where(sel, pden_ref[...], 0.0), axis=1)
        avgu = avgu_ref[...][:, 0]
        den_safe = jnp.where(tden == 0.0, 1.0, tden)
        pred = jnp.where(tden == 0.0, avgu, avgu + tnum / den_safe)
        out_ref[...] = pred[:, None]


def kernel(rating_matrix, user_embeddings, user_indices, item_indices):
    n_users, n_items = rating_matrix.shape
    d = user_embeddings.shape[1]
    b = user_indices.shape[0]
    nb = n_users // BN

    uidx_row = user_indices.reshape(1, b)
    uidx_col = user_indices.reshape(b, 1)
    iidx_col = item_indices.reshape(b, 1)

    u = pl.pallas_call(
        _gather_u_body,
        grid=(nb,),
        in_specs=[
            pl.BlockSpec((1, b), lambda i: (0, 0)),
            pl.BlockSpec((BN, d), lambda i: (i, 0)),
        ],
        out_specs=pl.BlockSpec((b, d), lambda i: (0, 0)),
        out_shape=jax.ShapeDtypeStruct((b, d), jnp.float32),
    )(uidx_row, user_embeddings)

    pred = pl.pallas_call(
        _main_body,
        grid=(nb,),
        in_specs=[
            pl.BlockSpec((b, 1), lambda i: (0, 0)),
            pl.BlockSpec((b, 1), lambda i: (0, 0)),
            pl.BlockSpec((b, d), lambda i: (0, 0)),
            pl.BlockSpec((BN, n_items), lambda i: (i, 0)),
            pl.BlockSpec((BN, d), lambda i: (i, 0)),
        ],
        out_specs=pl.BlockSpec((b, 1), lambda i: (0, 0)),
        out_shape=jax.ShapeDtypeStruct((b, 1), jnp.float32),
        scratch_shapes=[
            pltpu.VMEM((b, n_items), jnp.float32),
            pltpu.VMEM((b, n_items), jnp.float32),
            pltpu.VMEM((b, 1), jnp.float32),
        ],
    )(uidx_col, iidx_col, u, rating_matrix, user_embeddings)

    return pred.reshape(b)


# speedup vs baseline: 3.0261x; 3.0261x over previous
import jax
import jax.numpy as jnp
from jax.experimental import pallas as pl
from jax.experimental.pallas import tpu as pltpu

BN = 1024


def _gather_u_body(uidx_row_ref, e_ref, acc_ref):
    i = pl.program_id(0)
    bn, d = e_ref.shape
    b = uidx_row_ref.shape[1]
    m_ids = jax.lax.broadcasted_iota(jnp.int32, (bn, b), 0) + i * bn
    eq_t = (m_ids == uidx_row_ref[0, :][None, :]).astype(jnp.bfloat16)
    contrib = jax.lax.dot_general(
        eq_t, e_ref[...].astype(jnp.bfloat16),
        dimension_numbers=(((0,), (0,)), ((), ())),
        preferred_element_type=jnp.float32)

    @pl.when(i == 0)
    def _():
        acc_ref[...] = contrib

    @pl.when(i > 0)
    def _():
        acc_ref[...] += contrib


def _main_body(uidx_ref, iidx_ref, u_ref, r_ref, e_ref, out_ref,
               pnum_ref, pden_ref, avgu_ref, uhat_ref):
    i = pl.program_id(0)
    nb = pl.num_programs(0)
    bn, ni = r_ref.shape
    b = u_ref.shape[0]

    @pl.when(i == 0)
    def _():
        u = u_ref[...]
        nu2 = jnp.sum(u * u, axis=1)
        uhat_ref[...] = (u * jax.lax.rsqrt(jnp.maximum(nu2, 1e-60))[:, None]
                         ).astype(jnp.bfloat16)
        pnum_ref[...] = jnp.zeros_like(pnum_ref)
        pden_ref[...] = jnp.zeros_like(pden_ref)
        avgu_ref[...] = jnp.zeros_like(avgu_ref)

    r = r_ref[...]
    validb = jnp.logical_not(jnp.isnan(r))
    validf = validb.astype(jnp.float32)
    r0 = jnp.where(validb, r, 0.0)
    cnt = jnp.sum(validf, axis=1)
    ssum = jnp.sum(r0, axis=1)
    avg = jnp.where(cnt > 0.0, ssum / jnp.maximum(cnt, 1.0), 0.0)

    e = e_ref[...]
    nn2 = jnp.sum(e * e, axis=1)
    ehat = (e * jax.lax.rsqrt(jnp.maximum(nn2, 1e-60))[:, None]
            ).astype(jnp.bfloat16)

    sim = jax.lax.dot_general(
        uhat_ref[...], ehat,
        dimension_numbers=(((1,), (1,)), ((), ())),
        preferred_element_type=jnp.float32).astype(jnp.bfloat16)
    n_ids = jax.lax.broadcasted_iota(jnp.int32, (b, bn), 1) + i * bn
    eq = uidx_ref[...] == n_ids
    sim_x = jnp.where(eq, jnp.zeros_like(sim), sim)

    avgu_ref[...] += jnp.sum(jnp.where(eq, avg[None, :], 0.0), axis=1)[:, None]

    acomb = (r0 - avg[:, None] * validf).astype(jnp.bfloat16)
    pnum_ref[...] += jax.lax.dot_general(
        sim_x, acomb,
        dimension_numbers=(((1,), (0,)), ((), ())),
        preferred_element_type=jnp.float32)
    pden_ref[...] += jax.lax.dot_general(
        jnp.abs(sim_x), validf.astype(jnp.bfloat16),
        dimension_numbers=(((1,), (0,)), ((), ())),
        preferred_element_type=jnp.float32)

    @pl.when(i == nb - 1)
    def _():
        j_ids = jax.lax.broadcasted_iota(jnp.int32, (b, ni), 1)
        sel = iidx_ref[...] == j_ids
        tnum = jnp.sum(jnp.where(sel, pnum_ref[...], 0.0), axis=1)
        tden = jnp.sum(jnp.where(sel, pden_ref[...], 0.0), axis=1)
        avgu = avgu_ref[...][:, 0]
        den_safe = jnp.where(tden == 0.0, 1.0, tden)
        pred = jnp.where(tden == 0.0, avgu, avgu + tnum / den_safe)
        out_ref[...] = pred[:, None]


def kernel(rating_matrix, user_embeddings, user_indices, item_indices):
    n_users, n_items = rating_matrix.shape
    d = user_embeddings.shape[1]
    b = user_indices.shape[0]
    nb = n_users // BN

    uidx_row = user_indices.reshape(1, b)
    uidx_col = user_indices.reshape(b, 1)
    iidx_col = item_indices.reshape(b, 1)

    u = pl.pallas_call(
        _gather_u_body,
        grid=(nb,),
        in_specs=[
            pl.BlockSpec((1, b), lambda i: (0, 0)),
            pl.BlockSpec((BN, d), lambda i: (i, 0)),
        ],
        out_specs=pl.BlockSpec((b, d), lambda i: (0, 0)),
        out_shape=jax.ShapeDtypeStruct((b, d), jnp.float32),
    )(uidx_row, user_embeddings)

    pred = pl.pallas_call(
        _main_body,
        grid=(nb,),
        in_specs=[
            pl.BlockSpec((b, 1), lambda i: (0, 0)),
            pl.BlockSpec((b, 1), lambda i: (0, 0)),
            pl.BlockSpec((b, d), lambda i: (0, 0)),
            pl.BlockSpec((BN, n_items), lambda i: (i, 0)),
            pl.BlockSpec((BN, d), lambda i: (i, 0)),
        ],
        out_specs=pl.BlockSpec((b, 1), lambda i: (0, 0)),
        out_shape=jax.ShapeDtypeStruct((b, 1), jnp.float32),
        scratch_shapes=[
            pltpu.VMEM((b, n_items), jnp.float32),
            pltpu.VMEM((b, n_items), jnp.float32),
            pltpu.VMEM((b, 1), jnp.float32),
            pltpu.VMEM((b, d), jnp.bfloat16),
        ],
    )(uidx_col, iidx_col, u, rating_matrix, user_embeddings)

    return pred.reshape(b)
